# baseline (speedup 1.0000x reference)
import functools

import numpy as np
import jax
import jax.numpy as jnp
from jax import lax
from jax.experimental import pallas as pl
from jax.experimental.pallas import tpu as pltpu
from jax.experimental.pallas import tpu_sc as plsc

B, T, C = 1024, 50, 16
V, D = 1000000, 64
CV, CD = 128, 32
NF = 50
N = B * T


_NC, _NS = 2, 16
_NW = _NC * _NS
_BPW = N // _NW


def _sc_gather_call(table, idx1, idx2):
    mesh = plsc.VectorSubcoreMesh(core_axis_name="c", subcore_axis_name="s")

    @functools.partial(
        pl.kernel,
        mesh=mesh,
        compiler_params=pltpu.CompilerParams(use_tc_tiling_on_sc=False),
        out_type=(
            jax.ShapeDtypeStruct((N, 2 * D), jnp.float32),
            jax.ShapeDtypeStruct((N, 2 * D), jnp.float32),
        ),
        scratch_types=[
            pltpu.VMEM((_BPW // 2,), jnp.int32),
            pltpu.VMEM((_BPW // 2, 2 * D), jnp.float32),
            pltpu.SemaphoreType.DMA,
        ],
    )
    def k(table_hbm, idx1_hbm, idx2_hbm, out1_hbm, out2_hbm, idx_v, rows_v, sem):
        wid = lax.axis_index("s") * _NC + lax.axis_index("c")
        for idx_hbm, out_hbm in ((idx1_hbm, out1_hbm), (idx2_hbm, out2_hbm)):
            for h in range(2):
                base = wid * _BPW + h * (_BPW // 2)
                pltpu.sync_copy(idx_hbm.at[pl.ds(base, _BPW // 2)], idx_v)
                pltpu.async_copy(table_hbm.at[idx_v], rows_v, sem).wait()
                pltpu.sync_copy(rows_v, out_hbm.at[pl.ds(base, _BPW // 2)])

    return k(table, idx1, idx2)


_KS = (3, 4, 5)
_PS = tuple(C - K + 1 for K in _KS)
_NFP = 64
_AROWS = sum(P * _NFP for P in _PS)
_CE = 160


def _sel(K):
    P = C - K + 1
    s = np.zeros((C, P, K), np.float32)
    for p in range(P):
        for k in range(K):
            s[p + k, p, k] = 1.0
    return s


_SELS = tuple(_sel(K) for K in _KS)


def _build_AT(W3, W4, W5):
    blocks = []
    for W in (W3, W4, W5):
        K = W.shape[-1]
        P = C - K + 1
        Wp = jnp.pad(W, ((0, _NFP - NF), (0, 0), (0, 0)))
        rows = [jnp.pad(Wp, ((0, 0), (0, 0), (p, C - K - p)))
                for p in range(P)]
        blocks.append(jnp.stack(rows).reshape(-1, CD * C))
    return jnp.concatenate(blocks, axis=0)


def _char_body(qct_ref, ctbT_ref, AT_ref, bias_ref, ce_ref):
    qc = qct_ref[0]
    iota = lax.broadcasted_iota(jnp.int32, (CV, C, B), 0)
    oh = (iota == qc[None]).astype(jnp.bfloat16).reshape(CV * C, B)
    e = jnp.dot(ctbT_ref[...], oh,
                preferred_element_type=jnp.float32)
    y = jnp.dot(AT_ref[...], e.astype(jnp.bfloat16),
                preferred_element_type=jnp.float32
                ).astype(jnp.bfloat16)
    feats = []
    r0 = 0
    for P in _PS:
        m = y[r0:r0 + _NFP]
        for p in range(1, P):
            m = jnp.maximum(m, y[r0 + p * _NFP:r0 + (p + 1) * _NFP])
        feats.append(m[:NF])
        r0 += P * _NFP
    ce = jnp.concatenate(feats, axis=0).astype(jnp.float32)
    ce = ce + bias_ref[...][:3 * NF]
    ce = jnp.maximum(ce, 0.0)
    ce_ref[0, :3 * NF, :] = ce.astype(jnp.bfloat16)


def _char_call(qct, ctbT, AT, bias):
    return pl.pallas_call(
        _char_body,
        grid=(T,),
        in_specs=[
            pl.BlockSpec((1, C, B), lambda i: (i, 0, 0)),
            pl.BlockSpec((C * CD, CV * C), lambda i: (0, 0)),
            pl.BlockSpec((_AROWS, C * CD), lambda i: (0, 0)),
            pl.BlockSpec((_CE, 1), lambda i: (0, 0)),
        ],
        out_specs=pl.BlockSpec((1, _CE, B), lambda i: (i, 0, 0)),
        out_shape=jax.ShapeDtypeStruct((T, _CE, B), jnp.bfloat16),
    )(qct, ctbT, AT, bias)


_DTB = 2000


def _detile_body(a_ref, b_ref, o_ref):
    o_ref[...] = jnp.concatenate([a_ref[...], b_ref[...]], axis=1)


def _detile_call(word_table):
    nb = (V // 2) // _DTB
    return pl.pallas_call(
        _detile_body,
        grid=(nb,),
        in_specs=[
            pl.BlockSpec((_DTB, D), lambda i: (i, 0)),
            pl.BlockSpec((_DTB, D), lambda i: (i + (V // 2) // _DTB, 0)),
        ],
        out_specs=pl.BlockSpec((_DTB, 2 * D), lambda i: (i, 0)),
        out_shape=jax.ShapeDtypeStruct((V // 2, 2 * D), jnp.float32),
    )(word_table, word_table)


def _merge_body(wemb_ref, par_ref, ce_ref, out_ref):
    wpT = jnp.swapaxes(wemb_ref[...], 0, 1)
    par = par_ref[0]
    wt = jnp.where(par == 1, wpT[D:], wpT[:D])
    cb = ce_ref[0, :3 * NF, :].astype(jnp.float32)
    out_ref[0] = jnp.concatenate([wt, cb], axis=0)


def _merge_call(wemb, par, ce):
    return pl.pallas_call(
        _merge_body,
        grid=(T,),
        in_specs=[
            pl.BlockSpec((B, 2 * D), lambda i: (i, 0)),
            pl.BlockSpec((1, 1, B), lambda i: (i, 0, 0)),
            pl.BlockSpec((1, _CE, B), lambda i: (i, 0, 0)),
        ],
        out_specs=pl.BlockSpec((1, D + 3 * NF, B), lambda i: (i, 0, 0)),
        out_shape=jax.ShapeDtypeStruct((T, D + 3 * NF, B), jnp.float32),
    )(wemb, par, ce)


def kernel(word_table, char_table, W3, b3, W4, b4, W5, b5,
           q1, q2, q1_len, q2_len, q1_char, q2_char):
    AT = _build_AT(W3, W4, W5).astype(jnp.bfloat16)
    ctbT = jnp.kron(char_table.T,
                    jnp.eye(C, dtype=jnp.float32)).astype(jnp.bfloat16)
    bias = jnp.pad(jnp.concatenate([b3, b4, b5]), (0, _CE - 3 * NF))[:, None]
    qct1 = jnp.transpose(q1_char, (1, 2, 0))
    qct2 = jnp.transpose(q2_char, (1, 2, 0))
    ce1 = _char_call(qct1, ctbT, AT, bias)
    ce2 = _char_call(qct2, ctbT, AT, bias)

    idx1 = q1.T.reshape(-1).astype(jnp.int32)
    idx2 = q2.T.reshape(-1).astype(jnp.int32)
    table2 = _detile_call(word_table)
    wemb1, wemb2 = _sc_gather_call(table2, idx1 % (V // 2), idx2 % (V // 2))
    par1 = (idx1 // (V // 2)).reshape(T, 1, B)
    par2 = (idx2 // (V // 2)).reshape(T, 1, B)
    out1 = _merge_call(wemb1, par1, ce1)
    out2 = _merge_call(wemb2, par2, ce2)
    return (jnp.transpose(out1, (2, 0, 1)), jnp.transpose(out2, (2, 0, 1)))


# speedup vs baseline: 1.0150x; 1.0150x over previous
import functools

import numpy as np
import jax
import jax.numpy as jnp
from jax import lax
from jax.experimental import pallas as pl
from jax.experimental.pallas import tpu as pltpu
from jax.experimental.pallas import tpu_sc as plsc

B, T, C = 1024, 50, 16
V, D = 1000000, 64
CV, CD = 128, 32
NF = 50
N = B * T


_NC, _NS = 2, 16
_NW = _NC * _NS
_BPW = N // _NW


def _sc_gather_call(table, idx1, idx2):
    mesh = plsc.VectorSubcoreMesh(core_axis_name="c", subcore_axis_name="s")

    @functools.partial(
        pl.kernel,
        mesh=mesh,
        compiler_params=pltpu.CompilerParams(use_tc_tiling_on_sc=False),
        out_type=(
            jax.ShapeDtypeStruct((N, D), jnp.float32),
            jax.ShapeDtypeStruct((N, D), jnp.float32),
        ),
        scratch_types=[
            pltpu.VMEM((_BPW,), jnp.int32),
            pltpu.VMEM((_BPW, D), jnp.float32),
            pltpu.SemaphoreType.DMA,
        ],
    )
    def k(table_hbm, idx1_hbm, idx2_hbm, out1_hbm, out2_hbm, idx_v, rows_v, sem):
        wid = lax.axis_index("s") * _NC + lax.axis_index("c")
        base = wid * _BPW
        for idx_hbm, out_hbm in ((idx1_hbm, out1_hbm), (idx2_hbm, out2_hbm)):
            pltpu.sync_copy(idx_hbm.at[pl.ds(base, _BPW)], idx_v)
            pltpu.async_copy(table_hbm.at[idx_v], rows_v, sem).wait()
            pltpu.sync_copy(rows_v, out_hbm.at[pl.ds(base, _BPW)])

    return k(table, idx1, idx2)


_KS = (3, 4, 5)
_PS = tuple(C - K + 1 for K in _KS)
_NFP = 64
_AROWS = sum(P * _NFP for P in _PS)
_CE = 160


def _sel(K):
    P = C - K + 1
    s = np.zeros((C, P, K), np.float32)
    for p in range(P):
        for k in range(K):
            s[p + k, p, k] = 1.0
    return s


_SELS = tuple(_sel(K) for K in _KS)


def _build_AT(W3, W4, W5):
    blocks = []
    for W in (W3, W4, W5):
        K = W.shape[-1]
        P = C - K + 1
        Wp = jnp.pad(W, ((0, _NFP - NF), (0, 0), (0, 0)))
        rows = [jnp.pad(Wp, ((0, 0), (0, 0), (p, C - K - p)))
                for p in range(P)]
        blocks.append(jnp.stack(rows).reshape(-1, CD * C))
    return jnp.concatenate(blocks, axis=0)


def _char_body(qct_ref, ctbT_ref, AT_ref, bias_ref, ce_ref):
    qc = qct_ref[0]
    iota = lax.broadcasted_iota(jnp.int32, (CV, C, B), 0)
    oh = (iota == qc[None]).astype(jnp.bfloat16).reshape(CV * C, B)
    e = jnp.dot(ctbT_ref[...], oh,
                preferred_element_type=jnp.float32)
    y = jnp.dot(AT_ref[...], e.astype(jnp.bfloat16),
                preferred_element_type=jnp.float32
                ).astype(jnp.bfloat16)
    feats = []
    r0 = 0
    for P in _PS:
        m = y[r0:r0 + _NFP]
        for p in range(1, P):
            m = jnp.maximum(m, y[r0 + p * _NFP:r0 + (p + 1) * _NFP])
        feats.append(m[:NF])
        r0 += P * _NFP
    ce = jnp.concatenate(feats, axis=0).astype(jnp.float32)
    ce = ce + bias_ref[...][:3 * NF]
    ce = jnp.maximum(ce, 0.0)
    ce_ref[0, :3 * NF, :] = ce.astype(jnp.bfloat16)


def _char_call(qct, ctbT, AT, bias):
    return pl.pallas_call(
        _char_body,
        grid=(T,),
        in_specs=[
            pl.BlockSpec((1, C, B), lambda i: (i, 0, 0)),
            pl.BlockSpec((C * CD, CV * C), lambda i: (0, 0)),
            pl.BlockSpec((_AROWS, C * CD), lambda i: (0, 0)),
            pl.BlockSpec((_CE, 1), lambda i: (0, 0)),
        ],
        out_specs=pl.BlockSpec((1, _CE, B), lambda i: (i, 0, 0)),
        out_shape=jax.ShapeDtypeStruct((T, _CE, B), jnp.bfloat16),
    )(qct, ctbT, AT, bias)


def _merge_body(wemb_ref, ce_ref, out_ref):
    wt = jnp.swapaxes(wemb_ref[...], 0, 1)
    cb = ce_ref[0, :3 * NF, :].astype(jnp.float32)
    out_ref[0] = jnp.concatenate([wt, cb], axis=0)


def _merge_call(wemb, ce):
    return pl.pallas_call(
        _merge_body,
        grid=(T,),
        in_specs=[
            pl.BlockSpec((B, D), lambda i: (i, 0)),
            pl.BlockSpec((1, _CE, B), lambda i: (i, 0, 0)),
        ],
        out_specs=pl.BlockSpec((1, D + 3 * NF, B), lambda i: (i, 0, 0)),
        out_shape=jax.ShapeDtypeStruct((T, D + 3 * NF, B), jnp.float32),
    )(wemb, ce)


def kernel(word_table, char_table, W3, b3, W4, b4, W5, b5,
           q1, q2, q1_len, q2_len, q1_char, q2_char):
    AT = _build_AT(W3, W4, W5).astype(jnp.bfloat16)
    ctbT = jnp.kron(char_table.T,
                    jnp.eye(C, dtype=jnp.float32)).astype(jnp.bfloat16)
    bias = jnp.pad(jnp.concatenate([b3, b4, b5]), (0, _CE - 3 * NF))[:, None]
    qct1 = jnp.transpose(q1_char, (1, 2, 0))
    qct2 = jnp.transpose(q2_char, (1, 2, 0))
    ce1 = _char_call(qct1, ctbT, AT, bias)
    ce2 = _char_call(qct2, ctbT, AT, bias)

    idx1 = q1.T.reshape(-1).astype(jnp.int32)
    idx2 = q2.T.reshape(-1).astype(jnp.int32)
    wemb1, wemb2 = _sc_gather_call(word_table, idx1, idx2)
    out1 = _merge_call(wemb1, ce1)
    out2 = _merge_call(wemb2, ce2)
    return (jnp.transpose(out1, (2, 0, 1)), jnp.transpose(out2, (2, 0, 1)))
